# baseline (speedup 1.0000x reference)
import functools
import math

import jax
import jax.numpy as jnp
from jax import lax
from jax.experimental import pallas as pl
from jax.experimental.pallas import tpu as pltpu
from jax.experimental.pallas import tpu_sc as plsc

_NC = 2
_NS = 16
_NW = _NC * _NS
_L = 16

_CHUNK = 128


@functools.lru_cache(maxsize=None)
def _build(B, S, V, D):
    n_tb = B // _CHUNK
    n_s8 = S // 8
    scale = float(math.sqrt(D))
    mesh = plsc.VectorSubcoreMesh(
        core_axis_name="c", subcore_axis_name="s",
        num_cores=_NC, num_subcores=_NS)

    @functools.partial(
        pl.kernel,
        mesh=mesh,
        out_type=jax.ShapeDtypeStruct((S, D // 8, n_tb, 8, _CHUNK),
                                      jnp.float32),
        scratch_types=[
            pltpu.VMEM((n_s8, 1, 8, _CHUNK), jnp.int32),
            pltpu.VMEM((_CHUNK, D), jnp.float32),
            pltpu.VMEM((_CHUNK, D), jnp.float32),
            pltpu.VMEM((_CHUNK, D), jnp.float32),
            pltpu.VMEM((_CHUNK, D), jnp.float32),
            pltpu.VMEM((D // 8, 1, 8, _CHUNK), jnp.float32),
            pltpu.VMEM((D // 8, 1, 8, _CHUNK), jnp.float32),
            pltpu.VMEM((D // 8, 1, 8, _CHUNK), jnp.float32),
            pltpu.VMEM((D // 8, 1, 8, _CHUNK), jnp.float32),
            pltpu.SemaphoreType.DMA,
            pltpu.SemaphoreType.DMA,
        ],
        compiler_params=pltpu.CompilerParams(use_tc_tiling_on_sc=False,
                                             needs_layout_passes=False),
    )
    def gather_tr(tok_hbm, table_hbm, out_hbm, idx_v,
                  in0, in1, in2, in3, tr0, tr1, tr2, tr3, gsem, ssem):
        w = lax.axis_index("s") * _NC + lax.axis_index("c")

        pltpu.sync_copy(tok_hbm.at[:, pl.ds(w, 1)], idx_v)

        lanes = lax.iota(jnp.int32, _L)
        zero = lanes * 0
        rd_flat = []
        wr_flat = []
        for k in range(_L):
            d = lax.rem(lanes + k, _L)
            rd_flat.append(lanes * D + d)
            wr_flat.append(lax.div(d, 8) * (8 * _CHUNK)
                           + lax.rem(d, 8) * _CHUNK + lanes)

        def fire_gather(s, buf):
            return pltpu.async_copy(
                table_hbm.at[idx_v.at[s // 8, 0, s % 8]], buf, gsem)

        def drain_gather(buf):
            pltpu.make_async_copy(
                table_hbm.at[idx_v.at[0, 0, 0]], buf, gsem).wait()

        def transpose_scale(src, dst):
            def body(i, c):
                cb = i // (D // _L)
                eb = lax.rem(i, D // _L)
                rbase = cb * (_L * D) + eb * _L
                wbase = eb * (2 * 8 * _CHUNK) + cb * _L
                vs = [plsc.load_gather(src, [zero, rd_flat[k] + rbase])
                      for k in range(_L)]
                for k in range(_L):
                    plsc.store_scatter(
                        dst, [zero, zero, zero, wr_flat[k] + wbase],
                        vs[k] * scale)
                return c
            lax.fori_loop(0, (_CHUNK // _L) * (D // _L), body, 0)

        def out_slice(s):
            return out_hbm.at[s, pl.ds(0, D // 8), pl.ds(w, 1)]

        def drain_store(buf):
            pltpu.make_async_copy(buf, out_slice(0), ssem).wait()

        ins = (in0, in1, in2, in3)
        trs = (tr0, tr1, tr2, tr3)
        nb = len(ins)
        for b in range(nb):
            fire_gather(b, ins[b])

        def step(s, inb, trb):
            drain_gather(inb)
            @pl.when(s >= nb)
            def _():
                drain_store(trb)
            transpose_scale(inb, trb)
            pltpu.async_copy(trb, out_slice(s), ssem)
            @pl.when(s + nb < S)
            def _():
                fire_gather(s + nb, inb)

        def quad(i, carry):
            s0 = i * nb
            for b in range(nb):
                step(s0 + b, ins[b], trs[b])
            return carry

        lax.fori_loop(0, S // nb, quad, 0)

        for b in range(nb):
            drain_store(trs[b])

    return gather_tr


_DEPAD_BR = 2000


@functools.lru_cache(maxsize=None)
def _build_depad(V, D):
    def body(in_ref, out_ref):
        out_ref[:, :D] = in_ref[::2, :]
        out_ref[:, D:] = in_ref[1::2, :]

    return pl.pallas_call(
        body,
        grid=(V // _DEPAD_BR,),
        in_specs=[pl.BlockSpec((_DEPAD_BR, D), lambda i: (i, 0))],
        out_specs=pl.BlockSpec((_DEPAD_BR // 2, 2 * D), lambda i: (i, 0)),
        out_shape=jax.ShapeDtypeStruct((V // 2, 2 * D), jnp.float32),
    )


def kernel(tokens, weight):
    B, S = tokens.shape
    V, D = weight.shape
    tok4d = (tokens.astype(jnp.int32).T
             .reshape(S // 8, 8, B // _CHUNK, _CHUNK)
             .transpose(0, 2, 1, 3))
    table = _build_depad(V, D)(weight).reshape(V, D)
    out5d = _build(B, S, V, D)(tok4d, table)
    return out5d.transpose(2, 4, 0, 1, 3).reshape(B, S, D)


# speedup vs baseline: 1.2364x; 1.2364x over previous
import functools
import math

import jax
import jax.numpy as jnp
from jax import lax
from jax.experimental import pallas as pl
from jax.experimental.pallas import tpu as pltpu
from jax.experimental.pallas import tpu_sc as plsc

_NC = 2
_NS = 16
_NW = _NC * _NS
_L = 16

_CHUNK = 128


@functools.lru_cache(maxsize=None)
def _build(B, S, V, D):
    n_tb = B // _CHUNK
    n_s8 = S // 8
    scale = float(math.sqrt(D))
    mesh = plsc.VectorSubcoreMesh(
        core_axis_name="c", subcore_axis_name="s",
        num_cores=_NC, num_subcores=_NS)

    @functools.partial(
        pl.kernel,
        mesh=mesh,
        out_type=jax.ShapeDtypeStruct((S, D // 8, n_tb, 8, _CHUNK),
                                      jnp.float32),
        scratch_types=[
            pltpu.VMEM((n_s8, 1, 8, _CHUNK), jnp.int32),
            pltpu.VMEM((_CHUNK, D), jnp.float32),
            pltpu.VMEM((_CHUNK, D), jnp.float32),
            pltpu.VMEM((_CHUNK, D), jnp.float32),
            pltpu.VMEM((_CHUNK, D), jnp.float32),
            pltpu.VMEM((D // 8, 1, 8, _CHUNK), jnp.float32),
            pltpu.VMEM((D // 8, 1, 8, _CHUNK), jnp.float32),
            pltpu.VMEM((D // 8, 1, 8, _CHUNK), jnp.float32),
            pltpu.VMEM((D // 8, 1, 8, _CHUNK), jnp.float32),
            pltpu.SemaphoreType.DMA,
            pltpu.SemaphoreType.DMA,
        ],
        compiler_params=pltpu.CompilerParams(use_tc_tiling_on_sc=False,
                                             needs_layout_passes=False),
    )
    def gather_tr(tok_hbm, table_hbm, out_hbm, idx_v,
                  in0, in1, in2, in3, tr0, tr1, tr2, tr3, gsem, ssem):
        w = lax.axis_index("s") * _NC + lax.axis_index("c")

        pltpu.sync_copy(tok_hbm.at[:, pl.ds(w, 1)], idx_v)

        lanes = lax.iota(jnp.int32, _L)
        zero = lanes * 0
        rd_flat = []
        wr_flat = []
        for k in range(_L):
            d = lax.rem(lanes + k, _L)
            rd_flat.append(lanes * D + d)
            wr_flat.append(lax.div(d, 8) * (8 * _CHUNK)
                           + lax.rem(d, 8) * _CHUNK + lanes)

        def fire_gather(s, buf):
            return pltpu.async_copy(
                table_hbm.at[idx_v.at[s // 8, 0, s % 8]], buf, gsem)

        def drain_gather(buf):
            pltpu.make_async_copy(
                table_hbm.at[idx_v.at[0, 0, 0]], buf, gsem).wait()

        def transpose_scale(src, dst):
            def body(i, c):
                cb = i // (D // _L)
                eb = lax.rem(i, D // _L)
                rbase = cb * (_L * D) + eb * _L
                wbase = eb * (2 * 8 * _CHUNK) + cb * _L
                vs = [plsc.load_gather(src, [zero, rd_flat[k] + rbase])
                      for k in range(_L)]
                for k in range(_L):
                    plsc.store_scatter(
                        dst, [zero, zero, zero, wr_flat[k] + wbase],
                        vs[k] * scale)
                return c
            lax.fori_loop(0, (_CHUNK // _L) * (D // _L), body, 0)

        def out_slice(s):
            return out_hbm.at[s, pl.ds(0, D // 8), pl.ds(w, 1)]

        def drain_store(buf):
            pltpu.make_async_copy(buf, out_slice(0), ssem).wait()

        ins = (in0, in1, in2, in3)
        trs = (tr0, tr1, tr2, tr3)
        nb = len(ins)
        for b in range(nb):
            fire_gather(b, ins[b])

        def step(s, inb, trb):
            drain_gather(inb)
            @pl.when(s >= nb)
            def _():
                drain_store(trb)
            transpose_scale(inb, trb)
            pltpu.async_copy(trb, out_slice(s), ssem)
            @pl.when(s + nb < S)
            def _():
                fire_gather(s + nb, inb)

        def quad(i, carry):
            s0 = i * nb
            for b in range(nb):
                step(s0 + b, ins[b], trs[b])
            return carry

        lax.fori_loop(0, S // nb, quad, 0)

        for b in range(nb):
            drain_store(trs[b])

    return gather_tr


def kernel(tokens, weight):
    B, S = tokens.shape
    V, D = weight.shape
    tok4d = (tokens.astype(jnp.int32).T
             .reshape(S // 8, 8, B // _CHUNK, _CHUNK)
             .transpose(0, 2, 1, 3))
    out5d = _build(B, S, V, D)(tok4d, weight)
    return out5d.transpose(2, 4, 0, 1, 3).reshape(B, S, D)
